# baseline (speedup 1.0000x reference)
import functools

import jax
import jax.numpy as jnp
from jax import lax
from jax.experimental import pallas as pl
from jax.experimental.pallas import tpu as pltpu
from jax.experimental.pallas import tpu_sc as plsc

_N_ROWS = 131072
_N_COLS = 64
_NW = 32
_ROWS_PER_W = _N_ROWS // _NW
_CHUNK_ROWS = 512
_N_CHUNKS = _ROWS_PER_W // _CHUNK_ROWS

_PI_HI = 3.140625
_PI_LO = 9.67653589793e-4
_INV_PI = 0.3183098861837907
_HALF_PI = 1.5707963267948966

_S3 = -1.6666667163e-01
_S5 = 8.3333337680e-03
_S7 = -1.9841270114e-04

_SIGNBIT = jnp.int32(-2147483648)


def _f(v):
    return jnp.float32(v)


def _i32(x):
    return lax.bitcast_convert_type(x, jnp.int32)


def _f32(x):
    return lax.bitcast_convert_type(x, jnp.float32)


def _sin_poly_sc(t):
    u = t * _f(_INV_PI)
    h = _f32((_i32(u) & _SIGNBIT) | _i32(_f(0.5)))
    nr = (u + h).astype(jnp.int32)
    n = nr.astype(jnp.float32)
    odd = lax.shift_left(nr, 31)
    r = t - n * _f(_PI_HI)
    r = r - n * _f(_PI_LO)
    r2 = r * r
    p = _f(_S5) + r2 * _f(_S7)
    p = _f(_S3) + r2 * p
    p = r + r * (r2 * p)
    return _f32(_i32(p) ^ odd)


def _compute_vec(x, act, shift, c6, ce):
    ax = jnp.abs(x)
    relu = jnp.maximum(x, _f(0.0))
    earg = c6 * (x * x) + ce * ax
    e = jnp.exp(earg)
    den = _f(1.0) + e
    sig = jnp.where(x >= 0, _f(1.0), e) / den
    e2 = e * e
    mag = (_f(1.0) - e2) / (_f(1.0) + e2)
    th = _f32(_i32(mag) | (_i32(x) & _SIGNBIT))
    t = e / (_f(1.0) + den)
    t2 = t * t
    lp = _f(1 / 3) + t2 * _f(0.2)
    lp = _f(1.0) + t2 * lp
    sp = relu + _f(2.0) * t * lp
    s = _sin_poly_sc(x + shift)
    return jnp.where(act == 0, x,
           jnp.where(act == 1, relu,
           jnp.where(act == 2, sig,
           jnp.where(act == 3, th,
           jnp.where(act <= 5, s,
           jnp.where(act == 6, e,
           jnp.where(act == 7, ax, sp)))))))


def _sc_kernel(x_hbm, out_hbm, buf, tabi, tabf, sem):
    wid = lax.axis_index("s") * 2 + lax.axis_index("c")
    lane = lax.iota(jnp.int32, 16)
    for j in range(4):
        a = (lane + 16 * j) % 9
        tabi[j] = a
        tabf[j] = jnp.where(a == 5, _f(_HALF_PI), _f(0.0))
        tabf[4 + j] = jnp.where(a == 6, _f(-0.5), _f(0.0))
        tabf[8 + j] = jnp.where((a == 2) | (a == 3) | (a == 8), _f(-1.0), _f(0.0))

    def chunk_body(it, carry):
        base = wid * _ROWS_PER_W + it * _CHUNK_ROWS
        pltpu.sync_copy(x_hbm.at[pl.ds(base, _CHUNK_ROWS)], buf)

        @plsc.parallel_loop(0, _CHUNK_ROWS * 4, step=4, unroll=2)
        def vec_body(i):
            r = lax.shift_right_logical(i, 2)
            for p in range(4):
                v = buf[r, pl.ds(p * 16, 16)]
                buf[r, pl.ds(p * 16, 16)] = _compute_vec(
                    v, tabi[p], tabf[p], tabf[4 + p], tabf[8 + p])

        pltpu.sync_copy(buf, out_hbm.at[pl.ds(base, _CHUNK_ROWS)])
        return carry

    lax.fori_loop(0, _N_CHUNKS, chunk_body, 0)


@jax.jit
def kernel(x):
    mesh = plsc.VectorSubcoreMesh(core_axis_name="c", subcore_axis_name="s")
    return pl.kernel(
        _sc_kernel,
        mesh=mesh,
        out_type=jax.ShapeDtypeStruct((_N_ROWS, _N_COLS), jnp.float32),
        scratch_types=[
            pltpu.VMEM((_CHUNK_ROWS, _N_COLS), jnp.float32),
            pltpu.VMEM((4, 16), jnp.int32),
            pltpu.VMEM((12, 16), jnp.float32),
            pltpu.SemaphoreType.DMA,
        ],
    )(x)


# speedup vs baseline: 1.0750x; 1.0750x over previous
import functools

import jax
import jax.numpy as jnp
from jax import lax
from jax.experimental import pallas as pl
from jax.experimental.pallas import tpu as pltpu
from jax.experimental.pallas import tpu_sc as plsc

_N_ROWS = 131072
_N_COLS = 64
_NW = 32
_ROWS_PER_W = _N_ROWS // _NW
_CHUNK_ROWS = 512
_N_CHUNKS = _ROWS_PER_W // _CHUNK_ROWS

_PI_HI = 3.140625
_PI_LO = 9.67653589793e-4
_INV_PI = 0.3183098861837907
_HALF_PI = 1.5707963267948966

_S3 = -1.6666667163e-01
_S5 = 8.3333337680e-03
_S7 = -1.9841270114e-04

_SIGNBIT = jnp.int32(-2147483648)


def _f(v):
    return jnp.float32(v)


def _i32(x):
    return lax.bitcast_convert_type(x, jnp.int32)


def _f32(x):
    return lax.bitcast_convert_type(x, jnp.float32)


def _sin_poly_sc(t):
    u = t * _f(_INV_PI)
    h = _f32((_i32(u) & _SIGNBIT) | _i32(_f(0.5)))
    nr = (u + h).astype(jnp.int32)
    n = nr.astype(jnp.float32)
    odd = lax.shift_left(nr, 31)
    r = t - n * _f(_PI_HI)
    r = r - n * _f(_PI_LO)
    r2 = r * r
    p = _f(_S5) + r2 * _f(_S7)
    p = _f(_S3) + r2 * p
    p = r + r * (r2 * p)
    return _f32(_i32(p) ^ odd)


def _compute_vec(x, act, shift, c6, ce):
    ax = jnp.abs(x)
    relu = jnp.maximum(x, _f(0.0))
    earg = c6 * (x * x) + ce * ax
    e = jnp.exp(earg)
    den = _f(1.0) + e
    sig = jnp.where(x >= 0, _f(1.0), e) / den
    e2 = e * e
    mag = (_f(1.0) - e2) / (_f(1.0) + e2)
    th = _f32(_i32(mag) | (_i32(x) & _SIGNBIT))
    t = e / (_f(1.0) + den)
    t2 = t * t
    lp = _f(1 / 3) + t2 * _f(0.2)
    lp = _f(1.0) + t2 * lp
    sp = relu + _f(2.0) * t * lp
    s = _sin_poly_sc(x + shift)
    return jnp.where(act == 0, x,
           jnp.where(act == 1, relu,
           jnp.where(act == 2, sig,
           jnp.where(act == 3, th,
           jnp.where(act <= 5, s,
           jnp.where(act == 6, e,
           jnp.where(act == 7, ax, sp)))))))


def _sc_kernel(x_hbm, out_hbm, buf, tabi, tabf, sem):
    wid = lax.axis_index("s") * 2 + lax.axis_index("c")
    lane = lax.iota(jnp.int32, 16)
    for j in range(4):
        a = (lane + 16 * j) % 9
        tabi[j] = a
        tabf[j] = jnp.where(a == 5, _f(_HALF_PI), _f(0.0))
        tabf[4 + j] = jnp.where(a == 6, _f(-0.5), _f(0.0))
        tabf[8 + j] = jnp.where((a == 2) | (a == 3) | (a == 8), _f(-1.0), _f(0.0))

    def chunk_body(it, carry):
        base = wid * _ROWS_PER_W + it * _CHUNK_ROWS
        pltpu.sync_copy(x_hbm.at[pl.ds(base, _CHUNK_ROWS)], buf)

        @plsc.parallel_loop(0, _CHUNK_ROWS * 4, step=4)
        def vec_body(i):
            r = lax.shift_right_logical(i, 2)
            for p in range(4):
                v = buf[r, pl.ds(p * 16, 16)]
                buf[r, pl.ds(p * 16, 16)] = _compute_vec(
                    v, tabi[p], tabf[p], tabf[4 + p], tabf[8 + p])

        pltpu.sync_copy(buf, out_hbm.at[pl.ds(base, _CHUNK_ROWS)])
        return carry

    lax.fori_loop(0, _N_CHUNKS, chunk_body, 0)


@jax.jit
def kernel(x):
    mesh = plsc.VectorSubcoreMesh(core_axis_name="c", subcore_axis_name="s")
    return pl.kernel(
        _sc_kernel,
        mesh=mesh,
        out_type=jax.ShapeDtypeStruct((_N_ROWS, _N_COLS), jnp.float32),
        scratch_types=[
            pltpu.VMEM((_CHUNK_ROWS, _N_COLS), jnp.float32),
            pltpu.VMEM((4, 16), jnp.int32),
            pltpu.VMEM((12, 16), jnp.float32),
            pltpu.SemaphoreType.DMA,
        ],
    )(x)
